# baseline (speedup 1.0000x reference)
import functools

import jax
import jax.numpy as jnp
from jax import lax
from jax.experimental import pallas as pl
from jax.experimental.pallas import tpu as pltpu
from jax.experimental.pallas import tpu_sc as plsc

N = 10000
D = 128
E = 320000
NPAD = 10240
NW = 32
CH = 128
CPT = 80
EPT = CPT * CH
EPAD = NW * EPT
NBUF = 2
IB = 20
NIB = CPT // IB
ROWS_PER_SUB = NPAD // 16


def _aggregate(x_pad, src3, dst3):
    mesh = plsc.VectorSubcoreMesh(core_axis_name="c", subcore_axis_name="s")

    @functools.partial(
        pl.kernel,
        out_type=jax.ShapeDtypeStruct((2, NPAD, D), jnp.float32),
        mesh=mesh,
        scratch_types=[
            pltpu.VMEM((2, IB, CH), jnp.int32),
            pltpu.VMEM((2, IB, CH), jnp.int32),
            [pltpu.VMEM((CH, D), jnp.float32) for _ in range(NBUF)],
            pltpu.VMEM_SHARED((NPAD, D), jnp.float32),
            [pltpu.SemaphoreType.DMA for _ in range(NBUF)],
            [pltpu.SemaphoreType.DMA for _ in range(NBUF)],
            pltpu.SemaphoreType.DMA,
            pltpu.SemaphoreType.DMA,
            pltpu.SemaphoreType.DMA,
        ],
    )
    def agg(x_hbm, src_hbm, dst_hbm, out_hbm, src_v, dst_v, rows, acc,
            gsems, ssems, sisem, disem, asem):
        c = lax.axis_index("c")
        s = lax.axis_index("s")
        wid = s * 2 + c
        row0 = s * ROWS_PER_SUB

        init = pltpu.async_copy(x_hbm.at[pl.ds(row0, ROWS_PER_SUB)],
                                acc.at[pl.ds(row0, ROWS_PER_SUB)], asem)
        pltpu.async_copy(src_hbm.at[wid, 0], src_v.at[0], sisem)
        pltpu.async_copy(dst_hbm.at[wid, 0], dst_v.at[0], disem)
        init.wait()
        plsc.subcore_barrier()

        def blk_body(blk, carry):
            p = blk % 2
            pltpu.make_async_copy(src_hbm.at[wid, blk], src_v.at[p],
                                  sisem).wait()
            pltpu.make_async_copy(dst_hbm.at[wid, blk], dst_v.at[p],
                                  disem).wait()

            @pl.when(blk + 1 < NIB)
            def _():
                pltpu.async_copy(src_hbm.at[wid, blk + 1], src_v.at[1 - p],
                                 sisem)
                pltpu.async_copy(dst_hbm.at[wid, blk + 1], dst_v.at[1 - p],
                                 disem)

            for b in range(NBUF):
                pltpu.async_copy(x_hbm.at[src_v.at[p, b]], rows[b], gsems[b])
            for r in range(IB):
                b = r % NBUF
                o = (r + 1) % NBUF
                pltpu.make_async_copy(x_hbm.at[src_v.at[p, r]], rows[b],
                                      gsems[b]).wait()
                pltpu.async_copy(rows[b], acc.at[dst_v.at[p, r]], ssems[b],
                                 add=True)
                if r >= 1 and r + 1 < IB:
                    pltpu.make_async_copy(rows[o], acc.at[dst_v.at[p, r - 1]],
                                          ssems[o]).wait()
                    pltpu.async_copy(x_hbm.at[src_v.at[p, r + 1]], rows[o],
                                     gsems[o])
            for b in range(NBUF):
                pltpu.make_async_copy(rows[b], acc.at[dst_v.at[p, 0]],
                                      ssems[b]).wait()
            return carry

        lax.fori_loop(0, NIB, blk_body, 0)
        plsc.subcore_barrier()
        pltpu.sync_copy(acc.at[pl.ds(row0, ROWS_PER_SUB)],
                        out_hbm.at[c, pl.ds(row0, ROWS_PER_SUB)])

    return agg(x_pad, src3, dst3)


_BR = 256


def _mlp_body(p_ref, x_ref, wa, ba, wb, bb, o_ref):
    t = p_ref[0] + p_ref[1] - x_ref[...]
    h = jnp.maximum(
        jnp.dot(t, wa[...], preferred_element_type=jnp.float32) + ba[...], 0.0)
    o_ref[...] = jnp.dot(h, wb[...], preferred_element_type=jnp.float32) + bb[...]


def _mlp_head_body(p_ref, x_ref, wa, ba, wb, bb, wh, bh, o_ref):
    t = p_ref[0] + p_ref[1] - x_ref[...]
    h = jnp.maximum(
        jnp.dot(t, wa[...], preferred_element_type=jnp.float32) + ba[...], 0.0)
    g = jnp.dot(h, wb[...], preferred_element_type=jnp.float32) + bb[...]
    o_ref[...] = jnp.dot(g, wh[...], preferred_element_type=jnp.float32) + bh[...]


def _w_spec():
    return pl.BlockSpec((D, D), lambda i: (0, 0))


def _b_spec():
    return pl.BlockSpec((1, D), lambda i: (0, 0))


def _mlp(p, xin, Wa, ba, Wb, bb):
    return pl.pallas_call(
        _mlp_body,
        grid=(NPAD // _BR,),
        in_specs=[
            pl.BlockSpec((2, _BR, D), lambda i: (0, i, 0)),
            pl.BlockSpec((_BR, D), lambda i: (i, 0)),
            _w_spec(), _b_spec(), _w_spec(), _b_spec(),
        ],
        out_specs=pl.BlockSpec((_BR, D), lambda i: (i, 0)),
        out_shape=jax.ShapeDtypeStruct((NPAD, D), jnp.float32),
    )(p, xin, Wa, ba.reshape(1, D), Wb, bb.reshape(1, D))


def _mlp_head(p, xin, Wa, ba, Wb, bb, Wh, bh):
    return pl.pallas_call(
        _mlp_head_body,
        grid=(NPAD // _BR,),
        in_specs=[
            pl.BlockSpec((2, _BR, D), lambda i: (0, i, 0)),
            pl.BlockSpec((_BR, D), lambda i: (i, 0)),
            _w_spec(), _b_spec(), _w_spec(), _b_spec(), _w_spec(), _b_spec(),
        ],
        out_specs=pl.BlockSpec((_BR, D), lambda i: (i, 0)),
        out_shape=jax.ShapeDtypeStruct((NPAD, D), jnp.float32),
    )(p, xin, Wa, ba.reshape(1, D), Wb, bb.reshape(1, D), Wh, bh.reshape(1, D))


def kernel(x, edge_index, W1, b1, W2, b2, W3, b3, W4, b4, Wh, bh):
    x_pad = jnp.pad(x, ((0, NPAD - N), (0, 0)))
    fill = N + jnp.arange(EPAD - E, dtype=jnp.int32) % (NPAD - N)
    src3 = jnp.concatenate([edge_index[0], fill]).reshape(NW, NIB, IB, CH)
    dst3 = jnp.concatenate([edge_index[1], fill]).reshape(NW, NIB, IB, CH)

    p1 = _aggregate(x_pad, src3, dst3)
    h1 = _mlp(p1, x_pad, W1, b1, W2, b2)
    p2 = _aggregate(h1, src3, dst3)
    y = _mlp_head(p2, h1, W3, b3, W4, b4, Wh, bh)
    return y[:N]


# speedup vs baseline: 1.0360x; 1.0360x over previous
import functools

import jax
import jax.numpy as jnp
from jax import lax
from jax.experimental import pallas as pl
from jax.experimental.pallas import tpu as pltpu
from jax.experimental.pallas import tpu_sc as plsc

N = 10000
D = 128
E = 320000
NPAD = 10240
NW = 32
CH = 64
CPT = 160
EPT = CPT * CH
EPAD = NW * EPT
NBUF = 4
IB = 20
NIB = CPT // IB
ROWS_PER_SUB = NPAD // 16


def _aggregate(x_pad, src3, dst3):
    mesh = plsc.VectorSubcoreMesh(core_axis_name="c", subcore_axis_name="s")

    @functools.partial(
        pl.kernel,
        out_type=jax.ShapeDtypeStruct((2, NPAD, D), jnp.float32),
        mesh=mesh,
        scratch_types=[
            pltpu.VMEM((2, IB, CH), jnp.int32),
            pltpu.VMEM((2, IB, CH), jnp.int32),
            [pltpu.VMEM((CH, D), jnp.float32) for _ in range(NBUF)],
            pltpu.VMEM_SHARED((NPAD, D), jnp.float32),
            [pltpu.SemaphoreType.DMA for _ in range(NBUF)],
            [pltpu.SemaphoreType.DMA for _ in range(NBUF)],
            pltpu.SemaphoreType.DMA,
            pltpu.SemaphoreType.DMA,
            pltpu.SemaphoreType.DMA,
        ],
    )
    def agg(x_hbm, src_hbm, dst_hbm, out_hbm, src_v, dst_v, rows, acc,
            gsems, ssems, sisem, disem, asem):
        c = lax.axis_index("c")
        s = lax.axis_index("s")
        wid = s * 2 + c
        row0 = s * ROWS_PER_SUB

        init = pltpu.async_copy(x_hbm.at[pl.ds(row0, ROWS_PER_SUB)],
                                acc.at[pl.ds(row0, ROWS_PER_SUB)], asem)
        pltpu.async_copy(src_hbm.at[wid, 0], src_v.at[0], sisem)
        pltpu.async_copy(dst_hbm.at[wid, 0], dst_v.at[0], disem)
        init.wait()
        plsc.subcore_barrier()

        def blk_body(blk, carry):
            p = blk % 2
            pltpu.make_async_copy(src_hbm.at[wid, blk], src_v.at[p],
                                  sisem).wait()
            pltpu.make_async_copy(dst_hbm.at[wid, blk], dst_v.at[p],
                                  disem).wait()

            @pl.when(blk + 1 < NIB)
            def _():
                pltpu.async_copy(src_hbm.at[wid, blk + 1], src_v.at[1 - p],
                                 sisem)
                pltpu.async_copy(dst_hbm.at[wid, blk + 1], dst_v.at[1 - p],
                                 disem)

            for b in range(2):
                pltpu.async_copy(x_hbm.at[src_v.at[p, b]], rows[b], gsems[b])
            for r in range(IB):
                b = r % NBUF
                o = (r + 2) % NBUF
                pltpu.make_async_copy(x_hbm.at[src_v.at[p, r]], rows[b],
                                      gsems[b]).wait()
                pltpu.async_copy(rows[b], acc.at[dst_v.at[p, r]], ssems[b],
                                 add=True)
                if r + 2 < IB:
                    if r >= 2:
                        pltpu.make_async_copy(rows[o],
                                              acc.at[dst_v.at[p, r - 2]],
                                              ssems[o]).wait()
                    pltpu.async_copy(x_hbm.at[src_v.at[p, r + 2]], rows[o],
                                     gsems[o])
            for r in range(IB - NBUF, IB):
                b = r % NBUF
                pltpu.make_async_copy(rows[b], acc.at[dst_v.at[p, 0]],
                                      ssems[b]).wait()
            return carry

        lax.fori_loop(0, NIB, blk_body, 0)
        plsc.subcore_barrier()
        pltpu.sync_copy(acc.at[pl.ds(row0, ROWS_PER_SUB)],
                        out_hbm.at[c, pl.ds(row0, ROWS_PER_SUB)])

    return agg(x_pad, src3, dst3)


_BR = 256


def _mlp_body(p_ref, x_ref, wa, ba, wb, bb, o_ref):
    t = p_ref[0] + p_ref[1] - x_ref[...]
    h = jnp.maximum(
        jnp.dot(t, wa[...], preferred_element_type=jnp.float32) + ba[...], 0.0)
    o_ref[...] = jnp.dot(h, wb[...], preferred_element_type=jnp.float32) + bb[...]


def _mlp_head_body(p_ref, x_ref, wa, ba, wb, bb, wh, bh, o_ref):
    t = p_ref[0] + p_ref[1] - x_ref[...]
    h = jnp.maximum(
        jnp.dot(t, wa[...], preferred_element_type=jnp.float32) + ba[...], 0.0)
    g = jnp.dot(h, wb[...], preferred_element_type=jnp.float32) + bb[...]
    o_ref[...] = jnp.dot(g, wh[...], preferred_element_type=jnp.float32) + bh[...]


def _w_spec():
    return pl.BlockSpec((D, D), lambda i: (0, 0))


def _b_spec():
    return pl.BlockSpec((1, D), lambda i: (0, 0))


def _mlp(p, xin, Wa, ba, Wb, bb):
    return pl.pallas_call(
        _mlp_body,
        grid=(NPAD // _BR,),
        in_specs=[
            pl.BlockSpec((2, _BR, D), lambda i: (0, i, 0)),
            pl.BlockSpec((_BR, D), lambda i: (i, 0)),
            _w_spec(), _b_spec(), _w_spec(), _b_spec(),
        ],
        out_specs=pl.BlockSpec((_BR, D), lambda i: (i, 0)),
        out_shape=jax.ShapeDtypeStruct((NPAD, D), jnp.float32),
    )(p, xin, Wa, ba.reshape(1, D), Wb, bb.reshape(1, D))


def _mlp_head(p, xin, Wa, ba, Wb, bb, Wh, bh):
    return pl.pallas_call(
        _mlp_head_body,
        grid=(NPAD // _BR,),
        in_specs=[
            pl.BlockSpec((2, _BR, D), lambda i: (0, i, 0)),
            pl.BlockSpec((_BR, D), lambda i: (i, 0)),
            _w_spec(), _b_spec(), _w_spec(), _b_spec(), _w_spec(), _b_spec(),
        ],
        out_specs=pl.BlockSpec((_BR, D), lambda i: (i, 0)),
        out_shape=jax.ShapeDtypeStruct((NPAD, D), jnp.float32),
    )(p, xin, Wa, ba.reshape(1, D), Wb, bb.reshape(1, D), Wh, bh.reshape(1, D))


def kernel(x, edge_index, W1, b1, W2, b2, W3, b3, W4, b4, Wh, bh):
    x_pad = jnp.pad(x, ((0, NPAD - N), (0, 0)))
    fill = N + jnp.arange(EPAD - E, dtype=jnp.int32) % (NPAD - N)
    src3 = jnp.concatenate([edge_index[0], fill]).reshape(NW, NIB, IB, CH)
    dst3 = jnp.concatenate([edge_index[1], fill]).reshape(NW, NIB, IB, CH)

    p1 = _aggregate(x_pad, src3, dst3)
    h1 = _mlp(p1, x_pad, W1, b1, W2, b2)
    p2 = _aggregate(h1, src3, dst3)
    y = _mlp_head(p2, h1, W3, b3, W4, b4, Wh, bh)
    return y[:N]


# speedup vs baseline: 1.1401x; 1.1005x over previous
import functools

import jax
import jax.numpy as jnp
from jax import lax
from jax.experimental import pallas as pl
from jax.experimental.pallas import tpu as pltpu
from jax.experimental.pallas import tpu_sc as plsc

N = 10000
D = 128
E = 320000
NPAD = 10240
NW = 32
CH = 128
CPT = 80
EPT = CPT * CH
EPAD = NW * EPT
NBUF = 2
IB = 20
NIB = CPT // IB
ROWS_PER_SUB = NPAD // 16


def _aggregate(x_pad, src3, dst3):
    mesh = plsc.VectorSubcoreMesh(core_axis_name="c", subcore_axis_name="s")

    @functools.partial(
        pl.kernel,
        out_type=jax.ShapeDtypeStruct((2, NPAD, D), jnp.float32),
        mesh=mesh,
        scratch_types=[
            pltpu.VMEM((2, IB, CH), jnp.int32),
            pltpu.VMEM((2, IB, CH), jnp.int32),
            [pltpu.VMEM((CH, D), jnp.float32) for _ in range(NBUF)],
            pltpu.VMEM_SHARED((NPAD, D), jnp.float32),
            [pltpu.SemaphoreType.DMA for _ in range(NBUF)],
            [pltpu.SemaphoreType.DMA for _ in range(NBUF)],
            pltpu.SemaphoreType.DMA,
            pltpu.SemaphoreType.DMA,
            pltpu.SemaphoreType.DMA,
        ],
    )
    def agg(x_hbm, src_hbm, dst_hbm, out_hbm, src_v, dst_v, rows, acc,
            gsems, ssems, sisem, disem, asem):
        c = lax.axis_index("c")
        s = lax.axis_index("s")
        wid = s * 2 + c
        row0 = s * ROWS_PER_SUB

        init = pltpu.async_copy(x_hbm.at[pl.ds(row0, ROWS_PER_SUB)],
                                acc.at[pl.ds(row0, ROWS_PER_SUB)], asem)
        pltpu.async_copy(src_hbm.at[wid, 0], src_v.at[0], sisem)
        pltpu.async_copy(dst_hbm.at[wid, 0], dst_v.at[0], disem)
        init.wait()
        plsc.subcore_barrier()

        def blk_body(blk, carry):
            p = blk % 2
            pltpu.make_async_copy(src_hbm.at[wid, blk], src_v.at[p],
                                  sisem).wait()
            pltpu.make_async_copy(dst_hbm.at[wid, blk], dst_v.at[p],
                                  disem).wait()

            @pl.when(blk + 1 < NIB)
            def _():
                pltpu.async_copy(src_hbm.at[wid, blk + 1], src_v.at[1 - p],
                                 sisem)
                pltpu.async_copy(dst_hbm.at[wid, blk + 1], dst_v.at[1 - p],
                                 disem)

            for b in range(NBUF):
                pltpu.async_copy(x_hbm.at[src_v.at[p, b]], rows[b], gsems[b])
            for r in range(IB):
                b = r % NBUF
                pltpu.make_async_copy(x_hbm.at[src_v.at[p, r]], rows[b],
                                      gsems[b]).wait()
                pltpu.async_copy(rows[b], acc.at[dst_v.at[p, r]], ssems[b],
                                 add=True)
                if r + NBUF < IB:
                    pltpu.make_async_copy(rows[b], acc.at[dst_v.at[p, r]],
                                          ssems[b]).wait()
                    pltpu.async_copy(x_hbm.at[src_v.at[p, r + NBUF]], rows[b],
                                     gsems[b])
            for b in range(NBUF):
                pltpu.make_async_copy(rows[b], acc.at[dst_v.at[p, 0]],
                                      ssems[b]).wait()
            return carry

        lax.fori_loop(0, NIB, blk_body, 0)
        plsc.subcore_barrier()
        pltpu.sync_copy(acc.at[pl.ds(row0, ROWS_PER_SUB)],
                        out_hbm.at[c, pl.ds(row0, ROWS_PER_SUB)])

    return agg(x_pad, src3, dst3)


_BR = 256


def _mlp_body(p_ref, x_ref, wa, ba, wb, bb, o_ref):
    t = p_ref[0] + p_ref[1] - x_ref[...]
    h = jnp.maximum(
        jnp.dot(t, wa[...], preferred_element_type=jnp.float32) + ba[...], 0.0)
    o_ref[...] = jnp.dot(h, wb[...], preferred_element_type=jnp.float32) + bb[...]


def _mlp_head_body(p_ref, x_ref, wa, ba, wb, bb, wh, bh, o_ref):
    t = p_ref[0] + p_ref[1] - x_ref[...]
    h = jnp.maximum(
        jnp.dot(t, wa[...], preferred_element_type=jnp.float32) + ba[...], 0.0)
    g = jnp.dot(h, wb[...], preferred_element_type=jnp.float32) + bb[...]
    o_ref[...] = jnp.dot(g, wh[...], preferred_element_type=jnp.float32) + bh[...]


def _w_spec():
    return pl.BlockSpec((D, D), lambda i: (0, 0))


def _b_spec():
    return pl.BlockSpec((1, D), lambda i: (0, 0))


def _mlp(p, xin, Wa, ba, Wb, bb):
    return pl.pallas_call(
        _mlp_body,
        grid=(NPAD // _BR,),
        in_specs=[
            pl.BlockSpec((2, _BR, D), lambda i: (0, i, 0)),
            pl.BlockSpec((_BR, D), lambda i: (i, 0)),
            _w_spec(), _b_spec(), _w_spec(), _b_spec(),
        ],
        out_specs=pl.BlockSpec((_BR, D), lambda i: (i, 0)),
        out_shape=jax.ShapeDtypeStruct((NPAD, D), jnp.float32),
    )(p, xin, Wa, ba.reshape(1, D), Wb, bb.reshape(1, D))


def _mlp_head(p, xin, Wa, ba, Wb, bb, Wh, bh):
    return pl.pallas_call(
        _mlp_head_body,
        grid=(NPAD // _BR,),
        in_specs=[
            pl.BlockSpec((2, _BR, D), lambda i: (0, i, 0)),
            pl.BlockSpec((_BR, D), lambda i: (i, 0)),
            _w_spec(), _b_spec(), _w_spec(), _b_spec(), _w_spec(), _b_spec(),
        ],
        out_specs=pl.BlockSpec((_BR, D), lambda i: (i, 0)),
        out_shape=jax.ShapeDtypeStruct((NPAD, D), jnp.float32),
    )(p, xin, Wa, ba.reshape(1, D), Wb, bb.reshape(1, D), Wh, bh.reshape(1, D))


def kernel(x, edge_index, W1, b1, W2, b2, W3, b3, W4, b4, Wh, bh):
    x_pad = jnp.pad(x, ((0, NPAD - N), (0, 0)))
    fill = N + jnp.arange(EPAD - E, dtype=jnp.int32) % (NPAD - N)
    src3 = jnp.concatenate([edge_index[0], fill]).reshape(NW, NIB, IB, CH)
    dst3 = jnp.concatenate([edge_index[1], fill]).reshape(NW, NIB, IB, CH)

    p1 = _aggregate(x_pad, src3, dst3)
    h1 = _mlp(p1, x_pad, W1, b1, W2, b2)
    p2 = _aggregate(h1, src3, dst3)
    y = _mlp_head(p2, h1, W3, b3, W4, b4, Wh, bh)
    return y[:N]


# speedup vs baseline: 1.2955x; 1.1363x over previous
import functools

import jax
import jax.numpy as jnp
from jax import lax
from jax.experimental import pallas as pl
from jax.experimental.pallas import tpu as pltpu
from jax.experimental.pallas import tpu_sc as plsc

N = 10000
D = 128
E = 320000
NPAD = 10240
NW = 32
CH = 128
CPT = 80
EPT = CPT * CH
EPAD = NW * EPT
NBUF = 2
IB = 20
NIB = CPT // IB
ROWS_PER_SUB = NPAD // 16


def _aggregate(x_pad, src3, dst3):
    mesh = plsc.VectorSubcoreMesh(core_axis_name="c", subcore_axis_name="s")

    @functools.partial(
        pl.kernel,
        out_type=[jax.ShapeDtypeStruct((NPAD, D), jnp.float32),
                  jax.ShapeDtypeStruct((NPAD, D), jnp.float32)],
        mesh=mesh,
        scratch_types=[
            pltpu.VMEM((2, IB, CH), jnp.int32),
            pltpu.VMEM((2, IB, CH), jnp.int32),
            [pltpu.VMEM((CH, D), jnp.float32) for _ in range(NBUF)],
            pltpu.VMEM_SHARED((NPAD, D), jnp.float32),
            [pltpu.SemaphoreType.DMA for _ in range(NBUF)],
            [pltpu.SemaphoreType.DMA for _ in range(NBUF)],
            pltpu.SemaphoreType.DMA,
            pltpu.SemaphoreType.DMA,
            pltpu.SemaphoreType.DMA,
        ],
    )
    def agg(x_hbm, src_hbm, dst_hbm, out0_hbm, out1_hbm, src_v, dst_v, rows,
            acc, gsems, ssems, sisem, disem, asem):
        c = lax.axis_index("c")
        s = lax.axis_index("s")
        wid = s * 2 + c
        row0 = s * ROWS_PER_SUB

        init = pltpu.async_copy(x_hbm.at[pl.ds(row0, ROWS_PER_SUB)],
                                acc.at[pl.ds(row0, ROWS_PER_SUB)], asem)
        pltpu.async_copy(src_hbm.at[wid, 0], src_v.at[0], sisem)
        pltpu.async_copy(dst_hbm.at[wid, 0], dst_v.at[0], disem)
        init.wait()
        plsc.subcore_barrier()

        def blk_body(blk, carry):
            p = blk % 2
            pltpu.make_async_copy(src_hbm.at[wid, blk], src_v.at[p],
                                  sisem).wait()
            pltpu.make_async_copy(dst_hbm.at[wid, blk], dst_v.at[p],
                                  disem).wait()

            @pl.when(blk + 1 < NIB)
            def _():
                pltpu.async_copy(src_hbm.at[wid, blk + 1], src_v.at[1 - p],
                                 sisem)
                pltpu.async_copy(dst_hbm.at[wid, blk + 1], dst_v.at[1 - p],
                                 disem)

            for b in range(NBUF):
                pltpu.async_copy(x_hbm.at[src_v.at[p, b]], rows[b], gsems[b])
            for r in range(IB):
                b = r % NBUF
                pltpu.make_async_copy(x_hbm.at[src_v.at[p, r]], rows[b],
                                      gsems[b]).wait()
                pltpu.async_copy(rows[b], acc.at[dst_v.at[p, r]], ssems[b],
                                 add=True)
                if r + NBUF < IB:
                    pltpu.make_async_copy(rows[b], acc.at[dst_v.at[p, r]],
                                          ssems[b]).wait()
                    pltpu.async_copy(x_hbm.at[src_v.at[p, r + NBUF]], rows[b],
                                     gsems[b])
            for b in range(NBUF):
                pltpu.make_async_copy(rows[b], acc.at[dst_v.at[p, 0]],
                                      ssems[b]).wait()
            return carry

        lax.fori_loop(0, NIB, blk_body, 0)
        plsc.subcore_barrier()

        @pl.when(c == 0)
        def _():
            pltpu.sync_copy(acc.at[pl.ds(row0, ROWS_PER_SUB)],
                            out0_hbm.at[pl.ds(row0, ROWS_PER_SUB)])

        @pl.when(c == 1)
        def _():
            pltpu.sync_copy(acc.at[pl.ds(row0, ROWS_PER_SUB)],
                            out1_hbm.at[pl.ds(row0, ROWS_PER_SUB)])

    return agg(x_pad, src3, dst3)


_BR = 1024
_BRH = 1000


def _mlp_body(p0_ref, p1_ref, x_ref, wa, ba, wb, bb, o_ref):
    t = p0_ref[...] + p1_ref[...] - x_ref[...]
    h = jnp.maximum(
        jnp.dot(t, wa[...], preferred_element_type=jnp.float32) + ba[...], 0.0)
    o_ref[...] = jnp.dot(h, wb[...], preferred_element_type=jnp.float32) + bb[...]


def _mlp_head_body(p0_ref, p1_ref, x_ref, wa, ba, wb, bb, wh, bh, o_ref):
    t = p0_ref[...] + p1_ref[...] - x_ref[...]
    h = jnp.maximum(
        jnp.dot(t, wa[...], preferred_element_type=jnp.float32) + ba[...], 0.0)
    g = jnp.dot(h, wb[...], preferred_element_type=jnp.float32) + bb[...]
    o_ref[...] = jnp.dot(g, wh[...], preferred_element_type=jnp.float32) + bh[...]


def _w_spec():
    return pl.BlockSpec((D, D), lambda i: (0, 0))


def _b_spec():
    return pl.BlockSpec((1, D), lambda i: (0, 0))


def _row_spec(br):
    return pl.BlockSpec((br, D), lambda i: (i, 0))


def _mlp(p0, p1, xin, Wa, ba, Wb, bb):
    return pl.pallas_call(
        _mlp_body,
        grid=(NPAD // _BR,),
        in_specs=[
            _row_spec(_BR), _row_spec(_BR), _row_spec(_BR),
            _w_spec(), _b_spec(), _w_spec(), _b_spec(),
        ],
        out_specs=_row_spec(_BR),
        out_shape=jax.ShapeDtypeStruct((NPAD, D), jnp.float32),
    )(p0, p1, xin, Wa, ba.reshape(1, D), Wb, bb.reshape(1, D))


def _mlp_head(p0, p1, xin, Wa, ba, Wb, bb, Wh, bh):
    return pl.pallas_call(
        _mlp_head_body,
        grid=(N // _BRH,),
        in_specs=[
            _row_spec(_BRH), _row_spec(_BRH), _row_spec(_BRH),
            _w_spec(), _b_spec(), _w_spec(), _b_spec(), _w_spec(), _b_spec(),
        ],
        out_specs=_row_spec(_BRH),
        out_shape=jax.ShapeDtypeStruct((N, D), jnp.float32),
    )(p0, p1, xin, Wa, ba.reshape(1, D), Wb, bb.reshape(1, D), Wh,
      bh.reshape(1, D))


def kernel(x, edge_index, W1, b1, W2, b2, W3, b3, W4, b4, Wh, bh):
    x_pad = jnp.pad(x, ((0, NPAD - N), (0, 0)))
    fill = jnp.broadcast_to(N + jnp.arange(NPAD - N, dtype=jnp.int32),
                            ((EPAD - E) // (NPAD - N), NPAD - N)).reshape(-1)
    src3 = jnp.concatenate([edge_index[0], fill]).reshape(NW, NIB, IB, CH)
    dst3 = jnp.concatenate([edge_index[1], fill]).reshape(NW, NIB, IB, CH)

    p10, p11 = _aggregate(x_pad, src3, dst3)
    h1 = _mlp(p10, p11, x_pad, W1, b1, W2, b2)
    p20, p21 = _aggregate(h1, src3, dst3)
    return _mlp_head(p20, p21, h1, W3, b3, W4, b4, Wh, bh)


# speedup vs baseline: 1.3647x; 1.0534x over previous
import functools

import jax
import jax.numpy as jnp
from jax import lax
from jax.experimental import pallas as pl
from jax.experimental.pallas import tpu as pltpu
from jax.experimental.pallas import tpu_sc as plsc

N = 10000
D = 128
E = 320000
NW = 32
CH = 128
NBUF = 2
IB = 20
NB = E // (IB * CH)
RSUB = 624
RLAST = N - 15 * RSUB


def _aggregate(x, e3):
    mesh = plsc.VectorSubcoreMesh(core_axis_name="c", subcore_axis_name="s")

    @functools.partial(
        pl.kernel,
        out_type=[jax.ShapeDtypeStruct((N, D), jnp.float32),
                  jax.ShapeDtypeStruct((N, D), jnp.float32)],
        mesh=mesh,
        scratch_types=[
            pltpu.VMEM((2, IB, CH), jnp.int32),
            pltpu.VMEM((2, IB, CH), jnp.int32),
            [pltpu.VMEM((CH, D), jnp.float32) for _ in range(NBUF)],
            pltpu.VMEM_SHARED((N, D), jnp.float32),
            [pltpu.SemaphoreType.DMA for _ in range(NBUF)],
            [pltpu.SemaphoreType.DMA for _ in range(NBUF)],
            pltpu.SemaphoreType.DMA,
            pltpu.SemaphoreType.DMA,
            pltpu.SemaphoreType.DMA,
        ],
    )
    def agg(x_hbm, e_hbm, out0_hbm, out1_hbm, src_v, dst_v, rows, acc,
            gsems, ssems, sisem, disem, asem):
        c = lax.axis_index("c")
        s = lax.axis_index("s")
        wid = s * 2 + c
        row0 = s * RSUB
        blk0 = 4 * wid - jnp.maximum(wid - 29, 0)

        def idx_block(b, ring, half, sem):
            src = e_hbm.at[half, jnp.minimum(blk0 + b, NB - 1)]
            return pltpu.make_async_copy(src, ring.at[b % 2], sem)

        def run_block(b):
            p = b % 2
            idx_block(b, src_v, 0, sisem).wait()
            idx_block(b, dst_v, 1, disem).wait()
            if b + 1 < 4:
                idx_block(b + 1, src_v, 0, sisem).start()
                idx_block(b + 1, dst_v, 1, disem).start()
            for q in range(NBUF):
                pltpu.async_copy(x_hbm.at[src_v.at[p, q]], rows[q], gsems[q])
            for r in range(IB):
                q = r % NBUF
                pltpu.make_async_copy(x_hbm.at[src_v.at[p, r]], rows[q],
                                      gsems[q]).wait()
                pltpu.async_copy(rows[q], acc.at[dst_v.at[p, r]], ssems[q],
                                 add=True)
                if r + NBUF < IB:
                    pltpu.make_async_copy(rows[q], acc.at[dst_v.at[p, r]],
                                          ssems[q]).wait()
                    pltpu.async_copy(x_hbm.at[src_v.at[p, r + NBUF]], rows[q],
                                     gsems[q])
            for q in range(NBUF):
                pltpu.make_async_copy(rows[q], acc.at[dst_v.at[p, 0]],
                                      ssems[q]).wait()

        idx_block(0, src_v, 0, sisem).start()
        idx_block(0, dst_v, 1, disem).start()

        @pl.when(s < 15)
        def _():
            init = pltpu.async_copy(x_hbm.at[pl.ds(row0, RSUB)],
                                    acc.at[pl.ds(row0, RSUB)], asem)
            init.wait()

        @pl.when(s == 15)
        def _():
            init = pltpu.async_copy(x_hbm.at[pl.ds(15 * RSUB, RLAST)],
                                    acc.at[pl.ds(15 * RSUB, RLAST)], asem)
            init.wait()

        plsc.subcore_barrier()

        for b in range(3):
            run_block(b)

        @pl.when(wid < 29)
        def _():
            run_block(3)

        @pl.when(wid >= 29)
        def _():
            idx_block(3, src_v, 0, sisem).wait()
            idx_block(3, dst_v, 1, disem).wait()

        plsc.subcore_barrier()

        for cc, out_hbm in ((0, out0_hbm), (1, out1_hbm)):
            @pl.when((c == cc) & (s < 15))
            def _(out_hbm=out_hbm):
                pltpu.sync_copy(acc.at[pl.ds(row0, RSUB)],
                                out_hbm.at[pl.ds(row0, RSUB)])

            @pl.when((c == cc) & (s == 15))
            def _(out_hbm=out_hbm):
                pltpu.sync_copy(acc.at[pl.ds(15 * RSUB, RLAST)],
                                out_hbm.at[pl.ds(15 * RSUB, RLAST)])

    return agg(x, e3)


_BR = 1000


def _mlp_body(p0_ref, p1_ref, x_ref, wa, ba, wb, bb, o_ref):
    t = p0_ref[...] + p1_ref[...] - x_ref[...]
    h = jnp.maximum(
        jnp.dot(t, wa[...], preferred_element_type=jnp.float32) + ba[...], 0.0)
    o_ref[...] = jnp.dot(h, wb[...], preferred_element_type=jnp.float32) + bb[...]


def _mlp_head_body(p0_ref, p1_ref, x_ref, wa, ba, wb, bb, wh, bh, o_ref):
    t = p0_ref[...] + p1_ref[...] - x_ref[...]
    h = jnp.maximum(
        jnp.dot(t, wa[...], preferred_element_type=jnp.float32) + ba[...], 0.0)
    g = jnp.dot(h, wb[...], preferred_element_type=jnp.float32) + bb[...]
    o_ref[...] = jnp.dot(g, wh[...], preferred_element_type=jnp.float32) + bh[...]


def _w_spec():
    return pl.BlockSpec((D, D), lambda i: (0, 0))


def _b_spec():
    return pl.BlockSpec((1, D), lambda i: (0, 0))


def _row_spec():
    return pl.BlockSpec((_BR, D), lambda i: (i, 0))


def _mlp(p0, p1, xin, Wa, ba, Wb, bb):
    return pl.pallas_call(
        _mlp_body,
        grid=(N // _BR,),
        in_specs=[
            _row_spec(), _row_spec(), _row_spec(),
            _w_spec(), _b_spec(), _w_spec(), _b_spec(),
        ],
        out_specs=_row_spec(),
        out_shape=jax.ShapeDtypeStruct((N, D), jnp.float32),
    )(p0, p1, xin, Wa, ba.reshape(1, D), Wb, bb.reshape(1, D))


def _mlp_head(p0, p1, xin, Wa, ba, Wb, bb, Wh, bh):
    return pl.pallas_call(
        _mlp_head_body,
        grid=(N // _BR,),
        in_specs=[
            _row_spec(), _row_spec(), _row_spec(),
            _w_spec(), _b_spec(), _w_spec(), _b_spec(), _w_spec(), _b_spec(),
        ],
        out_specs=_row_spec(),
        out_shape=jax.ShapeDtypeStruct((N, D), jnp.float32),
    )(p0, p1, xin, Wa, ba.reshape(1, D), Wb, bb.reshape(1, D), Wh,
      bh.reshape(1, D))


def kernel(x, edge_index, W1, b1, W2, b2, W3, b3, W4, b4, Wh, bh):
    e3 = edge_index.reshape(2, NB, IB, CH)
    p10, p11 = _aggregate(x, e3)
    h1 = _mlp(p10, p11, x, W1, b1, W2, b2)
    p20, p21 = _aggregate(h1, e3)
    return _mlp_head(p20, p21, h1, W3, b3, W4, b4, Wh, bh)
